# baseline (speedup 1.0000x reference)
import functools

import jax
import jax.numpy as jnp
import numpy as np
from jax import lax
from jax.experimental import pallas as pl
from jax.experimental.pallas import tpu as pltpu
from jax.experimental.pallas import tpu_sc as plsc

PH = 7
PW = 7
SCALE = 64.0
H = 128
W = 128
C = 256
N_ROIS = 1000

L = 16
NW = 32
NPAD = 1024
R_PER_W = NPAD // NW
ROW_IDX = PW * L
BIN_ELEMS = PH * PW * C
CHUNKS = C // L
TR_BLK = 8


def _splat_lane(vec, lane, dtype=jnp.int32):
    dnums = lax.GatherDimensionNumbers(
        offset_dims=(), collapsed_slice_dims=(0,), start_index_map=(0,))
    idx = jnp.full((L, 1), lane, dtype)
    return lax.gather(vec, idx, dnums, (1,),
                      mode=lax.GatherScatterMode.PROMISE_IN_BOUNDS)


def _make_sc_kernel():
    mesh = plsc.VectorSubcoreMesh(core_axis_name="c", subcore_axis_name="s")

    @functools.partial(
        pl.kernel,
        out_type=jax.ShapeDtypeStruct((NPAD * BIN_ELEMS,), jnp.float32),
        mesh=mesh,
        scratch_types=[
            pltpu.VMEM((R_PER_W * 4 + L,), jnp.float32),
            pltpu.VMEM((ROW_IDX,), jnp.int32),
            pltpu.VMEM((ROW_IDX,), jnp.int32),
            pltpu.VMEM((ROW_IDX,), jnp.float32),
            pltpu.VMEM((ROW_IDX,), jnp.float32),
            pltpu.VMEM((ROW_IDX, C), jnp.float32),
            pltpu.VMEM((ROW_IDX, C), jnp.float32),
            pltpu.VMEM((BIN_ELEMS,), jnp.float32),
            pltpu.SemaphoreType.DMA,
            pltpu.SemaphoreType.DMA,
        ],
    )
    def roialign(fmap_hbm, rois_hbm, out_hbm, rois_v, idx0, idx1, w0, w1,
                 rows0, rows1, out_stage, sem0, sem1):
        wid = lax.axis_index("c") * 16 + lax.axis_index("s")
        idx_bufs = (idx0, idx1)
        w_bufs = (w0, w1)
        row_bufs = (rows0, rows1)
        sems = (sem0, sem1)

        pltpu.sync_copy(rois_hbm.at[pl.ds(wid * R_PER_W * 4, R_PER_W * 4)],
                        rois_v.at[pl.ds(0, R_PER_W * 4)])

        lane = lax.iota(jnp.int32, L)
        lane_syf = ((lane >> 3) & 1).astype(jnp.float32)
        lane_sxf = ((lane >> 2) & 1).astype(jnp.float32)
        lane_cy0 = ((lane >> 1) & 1) == 0
        lane_cx0 = (lane & 1) == 0

        def roi_body(r, carry):
            roi_chunk = rois_v[pl.ds(r * 4, L)]

            def splat(comp):
                return _splat_lane(roi_chunk, comp)

            x1s = splat(0) * SCALE
            y1s = splat(1) * SCALE
            x2s = splat(2) * SCALE
            y2s = splat(3) * SCALE
            roi_w = jnp.maximum(x2s - x1s, 1.0)
            roi_h = jnp.maximum(y2s - y1s, 1.0)
            bin_w = roi_w / float(PW)
            bin_h = roi_h / float(PH)

            def fill_row(ph):
                b = ph % 2
                ph_f = float(ph)

                def pw_body(pw, _):
                    pw_f = pw.astype(jnp.float32)
                    y = y1s + (ph_f + 0.25 + 0.5 * lane_syf) * bin_h
                    x = x1s + (pw_f + 0.25 + 0.5 * lane_sxf) * bin_w
                    valid = ((y > -1.0) & (y < float(H)) &
                             (x > -1.0) & (x < float(W)))
                    yc = jnp.minimum(jnp.maximum(y, 0.0), float(H - 1))
                    xc = jnp.minimum(jnp.maximum(x, 0.0), float(W - 1))
                    ylo = yc.astype(jnp.int32)
                    xlo = xc.astype(jnp.int32)
                    yhi = jnp.minimum(ylo + 1, H - 1)
                    xhi = jnp.minimum(xlo + 1, W - 1)
                    ly = yc - ylo.astype(jnp.float32)
                    lx = xc - xlo.astype(jnp.float32)
                    wy = jnp.where(lane_cy0, 1.0 - ly, ly)
                    wx = jnp.where(lane_cx0, 1.0 - lx, lx)
                    yi = jnp.where(lane_cy0, ylo, yhi)
                    xi = jnp.where(lane_cx0, xlo, xhi)
                    idx_bufs[b][pl.ds(pw * L, L)] = yi * W + xi
                    w_bufs[b][pl.ds(pw * L, L)] = (
                        wy * wx * jnp.where(valid, 0.25, 0.0))
                    return 0

                lax.fori_loop(0, PW, pw_body, 0)
                return pltpu.async_copy(fmap_hbm.at[idx_bufs[b]], row_bufs[b],
                                        sems[b])

            def combine_row(ph):
                b = ph % 2
                rows = row_bufs[b]
                wref = w_bufs[b]

                def pw_body(pw, _):
                    jbase = pw * L
                    wvec = wref[pl.ds(jbase, L)]

                    def j_body(j, acc):
                        wj = _splat_lane(wvec, j)
                        return tuple(
                            acc[k] + wj * rows[jbase + j, pl.ds(k * L, L)]
                            for k in range(CHUNKS))

                    acc = lax.fori_loop(
                        0, L, j_body,
                        tuple(jnp.zeros((L,), jnp.float32)
                              for _ in range(CHUNKS)))
                    obase = (ph * PW + pw) * C
                    for k in range(CHUNKS):
                        out_stage[pl.ds(obase + k * L, L)] = acc[k]
                    return 0

                lax.fori_loop(0, PW, pw_body, 0)

            cps = [None, None]
            for ph in range(PH):
                cps[ph % 2] = fill_row(ph)
                if ph > 0:
                    cps[(ph - 1) % 2].wait()
                    combine_row(ph - 1)
            cps[(PH - 1) % 2].wait()
            combine_row(PH - 1)

            out_base = (wid * R_PER_W + r) * BIN_ELEMS
            pltpu.sync_copy(out_stage,
                            out_hbm.at[pl.ds(out_base, BIN_ELEMS)])
            return carry

        lax.fori_loop(0, R_PER_W, roi_body, 0)

    return roialign


_SC_KERNEL = _make_sc_kernel()


def _tr_body(in_ref, out_ref):
    for i in range(TR_BLK):
        blk = in_ref[i]
        eye = (lax.broadcasted_iota(jnp.int32, (PH * PW, PH * PW), 0) ==
               lax.broadcasted_iota(jnp.int32, (PH * PW, PH * PW), 1)
               ).astype(jnp.float32)
        out_ref[i] = lax.dot_general(
            blk, eye, (((0,), (0,)), ((), ())),
            preferred_element_type=jnp.float32)


_TC_TRANSPOSE = pl.pallas_call(
    _tr_body,
    grid=(N_ROIS // TR_BLK,),
    in_specs=[pl.BlockSpec((TR_BLK, PH * PW, C), lambda i: (i, 0, 0))],
    out_specs=pl.BlockSpec((TR_BLK, C, PH * PW), lambda i: (i, 0, 0)),
    out_shape=jax.ShapeDtypeStruct((N_ROIS, C, PH * PW), jnp.float32),
)


@jax.jit
def kernel(features, rois):
    fmap = jnp.transpose(features, (0, 2, 3, 1)).reshape(H * W, C)
    rois_p = jnp.pad(rois, ((0, NPAD - N_ROIS), (0, 0))).reshape(NPAD * 4)
    out_flat = _SC_KERNEL(fmap, rois_p)
    out_nhwc = out_flat[:N_ROIS * BIN_ELEMS].reshape(N_ROIS, PH * PW, C)
    out = _TC_TRANSPOSE(out_nhwc)
    return out.reshape(N_ROIS, C, PH, PW)


# speedup vs baseline: 1.0803x; 1.0803x over previous
import functools

import jax
import jax.numpy as jnp
import numpy as np
from jax import lax
from jax.experimental import pallas as pl
from jax.experimental.pallas import tpu as pltpu
from jax.experimental.pallas import tpu_sc as plsc

PH = 7
PW = 7
SCALE = 64.0
H = 128
W = 128
C = 256
N_ROIS = 1000

L = 16
NW = 32
NPAD = 1024
R_PER_W = NPAD // NW
ROW_IDX = PW * L
BIN_ELEMS = PH * PW * C
CHUNKS = C // L
TR_BLK = 40


def _splat_lane(vec, lane, dtype=jnp.int32):
    dnums = lax.GatherDimensionNumbers(
        offset_dims=(), collapsed_slice_dims=(0,), start_index_map=(0,))
    idx = jnp.full((L, 1), lane, dtype)
    return lax.gather(vec, idx, dnums, (1,),
                      mode=lax.GatherScatterMode.PROMISE_IN_BOUNDS)


def _make_sc_kernel():
    mesh = plsc.VectorSubcoreMesh(core_axis_name="c", subcore_axis_name="s")

    @functools.partial(
        pl.kernel,
        out_type=jax.ShapeDtypeStruct((NPAD * BIN_ELEMS,), jnp.float32),
        mesh=mesh,
        scratch_types=[
            pltpu.VMEM((R_PER_W * 4 + L,), jnp.float32),
            pltpu.VMEM((ROW_IDX,), jnp.int32),
            pltpu.VMEM((ROW_IDX,), jnp.int32),
            pltpu.VMEM((ROW_IDX,), jnp.float32),
            pltpu.VMEM((ROW_IDX,), jnp.float32),
            pltpu.VMEM((ROW_IDX, C), jnp.float32),
            pltpu.VMEM((ROW_IDX, C), jnp.float32),
            pltpu.VMEM((BIN_ELEMS,), jnp.float32),
            pltpu.SemaphoreType.DMA,
            pltpu.SemaphoreType.DMA,
        ],
    )
    def roialign(fmap_hbm, rois_hbm, out_hbm, rois_v, idx0, idx1, w0, w1,
                 rows0, rows1, out_stage, sem0, sem1):
        wid = lax.axis_index("c") * 16 + lax.axis_index("s")
        idx_bufs = (idx0, idx1)
        w_bufs = (w0, w1)
        row_bufs = (rows0, rows1)
        sems = (sem0, sem1)

        pltpu.sync_copy(rois_hbm.at[pl.ds(wid * R_PER_W * 4, R_PER_W * 4)],
                        rois_v.at[pl.ds(0, R_PER_W * 4)])

        lane = lax.iota(jnp.int32, L)
        lane_syf = ((lane >> 3) & 1).astype(jnp.float32)
        lane_sxf = ((lane >> 2) & 1).astype(jnp.float32)
        lane_cy0 = ((lane >> 1) & 1) == 0
        lane_cx0 = (lane & 1) == 0

        def roi_body(r, carry):
            roi_chunk = rois_v[pl.ds(r * 4, L)]

            def splat(comp):
                return _splat_lane(roi_chunk, comp)

            x1s = splat(0) * SCALE
            y1s = splat(1) * SCALE
            x2s = splat(2) * SCALE
            y2s = splat(3) * SCALE
            roi_w = jnp.maximum(x2s - x1s, 1.0)
            roi_h = jnp.maximum(y2s - y1s, 1.0)
            bin_w = roi_w / float(PW)
            bin_h = roi_h / float(PH)

            def fill_row(ph):
                b = ph % 2
                ph_f = float(ph)

                def pw_body(pw, _):
                    pw_f = pw.astype(jnp.float32)
                    y = y1s + (ph_f + 0.25 + 0.5 * lane_syf) * bin_h
                    x = x1s + (pw_f + 0.25 + 0.5 * lane_sxf) * bin_w
                    valid = ((y > -1.0) & (y < float(H)) &
                             (x > -1.0) & (x < float(W)))
                    yc = jnp.minimum(jnp.maximum(y, 0.0), float(H - 1))
                    xc = jnp.minimum(jnp.maximum(x, 0.0), float(W - 1))
                    ylo = yc.astype(jnp.int32)
                    xlo = xc.astype(jnp.int32)
                    yhi = jnp.minimum(ylo + 1, H - 1)
                    xhi = jnp.minimum(xlo + 1, W - 1)
                    ly = yc - ylo.astype(jnp.float32)
                    lx = xc - xlo.astype(jnp.float32)
                    wy = jnp.where(lane_cy0, 1.0 - ly, ly)
                    wx = jnp.where(lane_cx0, 1.0 - lx, lx)
                    yi = jnp.where(lane_cy0, ylo, yhi)
                    xi = jnp.where(lane_cx0, xlo, xhi)
                    idx_bufs[b][pl.ds(pw * L, L)] = yi * W + xi
                    w_bufs[b][pl.ds(pw * L, L)] = (
                        wy * wx * jnp.where(valid, 0.25, 0.0))
                    return 0

                lax.fori_loop(0, PW, pw_body, 0)
                return pltpu.async_copy(fmap_hbm.at[idx_bufs[b]], row_bufs[b],
                                        sems[b])

            def combine_row(ph):
                b = ph % 2
                rows = row_bufs[b]
                wref = w_bufs[b]

                def pw_body(pw, _):
                    jbase = pw * L
                    wvec = wref[pl.ds(jbase, L)]

                    def j_body(j, acc):
                        wj = _splat_lane(wvec, j)
                        return tuple(
                            acc[k] + wj * rows[jbase + j, pl.ds(k * L, L)]
                            for k in range(CHUNKS))

                    acc = lax.fori_loop(
                        0, L, j_body,
                        tuple(jnp.zeros((L,), jnp.float32)
                              for _ in range(CHUNKS)))
                    obase = (ph * PW + pw) * C
                    for k in range(CHUNKS):
                        out_stage[pl.ds(obase + k * L, L)] = acc[k]
                    return 0

                lax.fori_loop(0, PW, pw_body, 0)

            cps = [None, None]
            for ph in range(PH):
                cps[ph % 2] = fill_row(ph)
                if ph > 0:
                    cps[(ph - 1) % 2].wait()
                    combine_row(ph - 1)
            cps[(PH - 1) % 2].wait()
            combine_row(PH - 1)

            out_base = (wid * R_PER_W + r) * BIN_ELEMS
            pltpu.sync_copy(out_stage,
                            out_hbm.at[pl.ds(out_base, BIN_ELEMS)])
            return carry

        lax.fori_loop(0, R_PER_W, roi_body, 0)

    return roialign


_SC_KERNEL = _make_sc_kernel()


def _tr_body(in_ref, out_ref):
    out_ref[...] = jnp.swapaxes(in_ref[...], 1, 2)


_TC_TRANSPOSE = pl.pallas_call(
    _tr_body,
    grid=(N_ROIS // TR_BLK,),
    in_specs=[pl.BlockSpec((TR_BLK, PH * PW, C), lambda i: (i, 0, 0))],
    out_specs=pl.BlockSpec((TR_BLK, C, PH * PW), lambda i: (i, 0, 0)),
    out_shape=jax.ShapeDtypeStruct((N_ROIS, C, PH * PW), jnp.float32),
)


@jax.jit
def kernel(features, rois):
    fmap = jnp.transpose(features, (0, 2, 3, 1)).reshape(H * W, C)
    rois_p = jnp.pad(rois, ((0, NPAD - N_ROIS), (0, 0))).reshape(NPAD * 4)
    out_flat = _SC_KERNEL(fmap, rois_p)
    out_nhwc = out_flat[:N_ROIS * BIN_ELEMS].reshape(N_ROIS, PH * PW, C)
    out = _TC_TRANSPOSE(out_nhwc)
    return out.reshape(N_ROIS, C, PH, PW)


# speedup vs baseline: 1.0912x; 1.0100x over previous
import functools

import jax
import jax.numpy as jnp
import numpy as np
from jax import lax
from jax.experimental import pallas as pl
from jax.experimental.pallas import tpu as pltpu
from jax.experimental.pallas import tpu_sc as plsc

PH = 7
PW = 7
SCALE = 64.0
H = 128
W = 128
C = 256
N_ROIS = 1000

L = 16
NW = 32
NPAD = 1024
R_PER_W = NPAD // NW
ROW_IDX = PW * L
BIN_ELEMS = PH * PW * C
CHUNKS = C // L
TR_BLK = 40


def _splat_lane(vec, lane, dtype=jnp.int32):
    dnums = lax.GatherDimensionNumbers(
        offset_dims=(), collapsed_slice_dims=(0,), start_index_map=(0,))
    idx = jnp.full((L, 1), lane, dtype)
    return lax.gather(vec, idx, dnums, (1,),
                      mode=lax.GatherScatterMode.PROMISE_IN_BOUNDS)


def _make_sc_kernel():
    mesh = plsc.VectorSubcoreMesh(core_axis_name="c", subcore_axis_name="s")

    @functools.partial(
        pl.kernel,
        out_type=jax.ShapeDtypeStruct((NPAD * BIN_ELEMS,), jnp.float32),
        mesh=mesh,
        scratch_types=[
            pltpu.VMEM((R_PER_W * 4 + L,), jnp.float32),
            pltpu.VMEM((ROW_IDX,), jnp.int32),
            pltpu.VMEM((ROW_IDX,), jnp.int32),
            pltpu.VMEM((ROW_IDX,), jnp.int32),
            pltpu.VMEM((ROW_IDX,), jnp.float32),
            pltpu.VMEM((ROW_IDX,), jnp.float32),
            pltpu.VMEM((ROW_IDX,), jnp.float32),
            pltpu.VMEM((ROW_IDX, C), jnp.float32),
            pltpu.VMEM((ROW_IDX, C), jnp.float32),
            pltpu.VMEM((ROW_IDX, C), jnp.float32),
            pltpu.VMEM((BIN_ELEMS,), jnp.float32),
            pltpu.SemaphoreType.DMA,
            pltpu.SemaphoreType.DMA,
            pltpu.SemaphoreType.DMA,
        ],
    )
    def roialign(fmap_hbm, rois_hbm, out_hbm, rois_v, idx0, idx1, idx2,
                 w0, w1, w2, rows0, rows1, rows2, out_stage,
                 sem0, sem1, sem2):
        wid = lax.axis_index("c") * 16 + lax.axis_index("s")
        idx_bufs = (idx0, idx1, idx2)
        w_bufs = (w0, w1, w2)
        row_bufs = (rows0, rows1, rows2)
        sems = (sem0, sem1, sem2)

        pltpu.sync_copy(rois_hbm.at[pl.ds(wid * R_PER_W * 4, R_PER_W * 4)],
                        rois_v.at[pl.ds(0, R_PER_W * 4)])

        lane = lax.iota(jnp.int32, L)
        lane_syf = ((lane >> 3) & 1).astype(jnp.float32)
        lane_sxf = ((lane >> 2) & 1).astype(jnp.float32)
        lane_cy0 = ((lane >> 1) & 1) == 0
        lane_cx0 = (lane & 1) == 0

        def roi_body(r, carry):
            roi_chunk = rois_v[pl.ds(r * 4, L)]

            def splat(comp):
                return _splat_lane(roi_chunk, comp)

            x1s = splat(0) * SCALE
            y1s = splat(1) * SCALE
            x2s = splat(2) * SCALE
            y2s = splat(3) * SCALE
            roi_w = jnp.maximum(x2s - x1s, 1.0)
            roi_h = jnp.maximum(y2s - y1s, 1.0)
            bin_w = roi_w / float(PW)
            bin_h = roi_h / float(PH)

            def fill_row(ph):
                b = ph % 3
                ph_f = float(ph)

                def pw_body(pw, _):
                    pw_f = pw.astype(jnp.float32)
                    y = y1s + (ph_f + 0.25 + 0.5 * lane_syf) * bin_h
                    x = x1s + (pw_f + 0.25 + 0.5 * lane_sxf) * bin_w
                    valid = ((y > -1.0) & (y < float(H)) &
                             (x > -1.0) & (x < float(W)))
                    yc = jnp.minimum(jnp.maximum(y, 0.0), float(H - 1))
                    xc = jnp.minimum(jnp.maximum(x, 0.0), float(W - 1))
                    ylo = yc.astype(jnp.int32)
                    xlo = xc.astype(jnp.int32)
                    yhi = jnp.minimum(ylo + 1, H - 1)
                    xhi = jnp.minimum(xlo + 1, W - 1)
                    ly = yc - ylo.astype(jnp.float32)
                    lx = xc - xlo.astype(jnp.float32)
                    wy = jnp.where(lane_cy0, 1.0 - ly, ly)
                    wx = jnp.where(lane_cx0, 1.0 - lx, lx)
                    yi = jnp.where(lane_cy0, ylo, yhi)
                    xi = jnp.where(lane_cx0, xlo, xhi)
                    idx_bufs[b][pl.ds(pw * L, L)] = yi * W + xi
                    w_bufs[b][pl.ds(pw * L, L)] = (
                        wy * wx * jnp.where(valid, 0.25, 0.0))
                    return 0

                lax.fori_loop(0, PW, pw_body, 0)
                return pltpu.async_copy(fmap_hbm.at[idx_bufs[b]], row_bufs[b],
                                        sems[b])

            def combine_row(ph):
                b = ph % 3
                rows = row_bufs[b]
                wref = w_bufs[b]

                def pw_body(pw, _):
                    jbase = pw * L
                    wvec = wref[pl.ds(jbase, L)]

                    def j_body(j, acc):
                        wj = _splat_lane(wvec, j)
                        return tuple(
                            acc[k] + wj * rows[jbase + j, pl.ds(k * L, L)]
                            for k in range(CHUNKS))

                    acc = lax.fori_loop(
                        0, L, j_body,
                        tuple(jnp.zeros((L,), jnp.float32)
                              for _ in range(CHUNKS)))
                    obase = (ph * PW + pw) * C
                    for k in range(CHUNKS):
                        out_stage[pl.ds(obase + k * L, L)] = acc[k]
                    return 0

                lax.fori_loop(0, PW, pw_body, 0)

            cps = [None, None, None]
            for ph in range(PH):
                cps[ph % 3] = fill_row(ph)
                if ph >= 2:
                    cps[(ph - 2) % 3].wait()
                    combine_row(ph - 2)
            for ph in (PH - 2, PH - 1):
                cps[ph % 3].wait()
                combine_row(ph)

            out_base = (wid * R_PER_W + r) * BIN_ELEMS
            pltpu.sync_copy(out_stage,
                            out_hbm.at[pl.ds(out_base, BIN_ELEMS)])
            return carry

        lax.fori_loop(0, R_PER_W, roi_body, 0)

    return roialign


_SC_KERNEL = _make_sc_kernel()


def _tr_body(in_ref, out_ref):
    out_ref[...] = jnp.swapaxes(in_ref[...], 1, 2)


_TC_TRANSPOSE = pl.pallas_call(
    _tr_body,
    grid=(N_ROIS // TR_BLK,),
    in_specs=[pl.BlockSpec((TR_BLK, PH * PW, C), lambda i: (i, 0, 0))],
    out_specs=pl.BlockSpec((TR_BLK, C, PH * PW), lambda i: (i, 0, 0)),
    out_shape=jax.ShapeDtypeStruct((N_ROIS, C, PH * PW), jnp.float32),
)


@jax.jit
def kernel(features, rois):
    fmap = jnp.transpose(features, (0, 2, 3, 1)).reshape(H * W, C)
    rois_p = jnp.pad(rois, ((0, NPAD - N_ROIS), (0, 0))).reshape(NPAD * 4)
    out_flat = _SC_KERNEL(fmap, rois_p)
    out_nhwc = out_flat[:N_ROIS * BIN_ELEMS].reshape(N_ROIS, PH * PW, C)
    out = _TC_TRANSPOSE(out_nhwc)
    return out.reshape(N_ROIS, C, PH, PW)


# speedup vs baseline: 1.0931x; 1.0017x over previous
import functools

import jax
import jax.numpy as jnp
import numpy as np
from jax import lax
from jax.experimental import pallas as pl
from jax.experimental.pallas import tpu as pltpu
from jax.experimental.pallas import tpu_sc as plsc

PH = 7
PW = 7
SCALE = 64.0
H = 128
W = 128
C = 256
N_ROIS = 1000

L = 16
NW = 32
N_HALF = N_ROIS // 2
NPAD = 512
R_PER_W = NPAD // NW
ROW_IDX = PW * L
BIN_ELEMS = PH * PW * C
CHUNKS = C // L
TR_BLK = 50


def _splat_lane(vec, lane, dtype=jnp.int32):
    dnums = lax.GatherDimensionNumbers(
        offset_dims=(), collapsed_slice_dims=(0,), start_index_map=(0,))
    idx = jnp.full((L, 1), lane, dtype)
    return lax.gather(vec, idx, dnums, (1,),
                      mode=lax.GatherScatterMode.PROMISE_IN_BOUNDS)


def _make_sc_kernel():
    mesh = plsc.VectorSubcoreMesh(core_axis_name="c", subcore_axis_name="s")

    @functools.partial(
        pl.kernel,
        out_type=jax.ShapeDtypeStruct((NPAD * BIN_ELEMS,), jnp.float32),
        mesh=mesh,
        scratch_types=[
            pltpu.VMEM((R_PER_W * 4 + L,), jnp.float32),
            pltpu.VMEM((ROW_IDX,), jnp.int32),
            pltpu.VMEM((ROW_IDX,), jnp.int32),
            pltpu.VMEM((ROW_IDX,), jnp.int32),
            pltpu.VMEM((ROW_IDX,), jnp.float32),
            pltpu.VMEM((ROW_IDX,), jnp.float32),
            pltpu.VMEM((ROW_IDX,), jnp.float32),
            pltpu.VMEM((ROW_IDX, C), jnp.float32),
            pltpu.VMEM((ROW_IDX, C), jnp.float32),
            pltpu.VMEM((ROW_IDX, C), jnp.float32),
            pltpu.VMEM((BIN_ELEMS,), jnp.float32),
            pltpu.SemaphoreType.DMA,
            pltpu.SemaphoreType.DMA,
            pltpu.SemaphoreType.DMA,
        ],
    )
    def roialign(fmap_hbm, rois_hbm, out_hbm, rois_v, idx0, idx1, idx2,
                 w0, w1, w2, rows0, rows1, rows2, out_stage,
                 sem0, sem1, sem2):
        wid = lax.axis_index("c") * 16 + lax.axis_index("s")
        idx_bufs = (idx0, idx1, idx2)
        w_bufs = (w0, w1, w2)
        row_bufs = (rows0, rows1, rows2)
        sems = (sem0, sem1, sem2)

        pltpu.sync_copy(rois_hbm.at[pl.ds(wid * R_PER_W * 4, R_PER_W * 4)],
                        rois_v.at[pl.ds(0, R_PER_W * 4)])

        lane = lax.iota(jnp.int32, L)
        lane_syf = ((lane >> 3) & 1).astype(jnp.float32)
        lane_sxf = ((lane >> 2) & 1).astype(jnp.float32)
        lane_cy0 = ((lane >> 1) & 1) == 0
        lane_cx0 = (lane & 1) == 0

        def roi_body(r, carry):
            roi_chunk = rois_v[pl.ds(r * 4, L)]

            def splat(comp):
                return _splat_lane(roi_chunk, comp)

            x1s = splat(0) * SCALE
            y1s = splat(1) * SCALE
            x2s = splat(2) * SCALE
            y2s = splat(3) * SCALE
            roi_w = jnp.maximum(x2s - x1s, 1.0)
            roi_h = jnp.maximum(y2s - y1s, 1.0)
            bin_w = roi_w / float(PW)
            bin_h = roi_h / float(PH)

            def fill_row(ph):
                b = ph % 3
                ph_f = float(ph)

                def pw_body(pw, _):
                    pw_f = pw.astype(jnp.float32)
                    y = y1s + (ph_f + 0.25 + 0.5 * lane_syf) * bin_h
                    x = x1s + (pw_f + 0.25 + 0.5 * lane_sxf) * bin_w
                    valid = ((y > -1.0) & (y < float(H)) &
                             (x > -1.0) & (x < float(W)))
                    yc = jnp.minimum(jnp.maximum(y, 0.0), float(H - 1))
                    xc = jnp.minimum(jnp.maximum(x, 0.0), float(W - 1))
                    ylo = yc.astype(jnp.int32)
                    xlo = xc.astype(jnp.int32)
                    yhi = jnp.minimum(ylo + 1, H - 1)
                    xhi = jnp.minimum(xlo + 1, W - 1)
                    ly = yc - ylo.astype(jnp.float32)
                    lx = xc - xlo.astype(jnp.float32)
                    wy = jnp.where(lane_cy0, 1.0 - ly, ly)
                    wx = jnp.where(lane_cx0, 1.0 - lx, lx)
                    yi = jnp.where(lane_cy0, ylo, yhi)
                    xi = jnp.where(lane_cx0, xlo, xhi)
                    idx_bufs[b][pl.ds(pw * L, L)] = yi * W + xi
                    w_bufs[b][pl.ds(pw * L, L)] = (
                        wy * wx * jnp.where(valid, 0.25, 0.0))
                    return 0

                lax.fori_loop(0, PW, pw_body, 0)
                return pltpu.async_copy(fmap_hbm.at[idx_bufs[b]], row_bufs[b],
                                        sems[b])

            def combine_row(ph):
                b = ph % 3
                rows = row_bufs[b]
                wref = w_bufs[b]

                def pw_body(pw, _):
                    jbase = pw * L
                    wvec = wref[pl.ds(jbase, L)]

                    def j_body(j, acc):
                        wj = _splat_lane(wvec, j)
                        return tuple(
                            acc[k] + wj * rows[jbase + j, pl.ds(k * L, L)]
                            for k in range(CHUNKS))

                    acc = lax.fori_loop(
                        0, L, j_body,
                        tuple(jnp.zeros((L,), jnp.float32)
                              for _ in range(CHUNKS)))
                    obase = (ph * PW + pw) * C
                    for k in range(CHUNKS):
                        out_stage[pl.ds(obase + k * L, L)] = acc[k]
                    return 0

                lax.fori_loop(0, PW, pw_body, 0)

            cps = [None, None, None]
            for ph in range(PH):
                cps[ph % 3] = fill_row(ph)
                if ph >= 2:
                    cps[(ph - 2) % 3].wait()
                    combine_row(ph - 2)
            for ph in (PH - 2, PH - 1):
                cps[ph % 3].wait()
                combine_row(ph)

            out_base = (wid * R_PER_W + r) * BIN_ELEMS
            pltpu.sync_copy(out_stage,
                            out_hbm.at[pl.ds(out_base, BIN_ELEMS)])
            return carry

        lax.fori_loop(0, R_PER_W, roi_body, 0)

    return roialign


_SC_KERNEL = _make_sc_kernel()


def _tr_body(in_ref, out_ref):
    out_ref[...] = jnp.swapaxes(in_ref[...], 1, 2)


_TC_TRANSPOSE = pl.pallas_call(
    _tr_body,
    grid=(N_HALF // TR_BLK,),
    in_specs=[pl.BlockSpec((TR_BLK, PH * PW, C), lambda i: (i, 0, 0))],
    out_specs=pl.BlockSpec((TR_BLK, C, PH * PW), lambda i: (i, 0, 0)),
    out_shape=jax.ShapeDtypeStruct((N_HALF, C, PH * PW), jnp.float32),
)


@jax.jit
def kernel(features, rois):
    fmap = jnp.transpose(features, (0, 2, 3, 1)).reshape(H * W, C)
    halves = []
    for h in range(2):
        rp = jnp.pad(rois[h * N_HALF:(h + 1) * N_HALF],
                     ((0, NPAD - N_HALF), (0, 0))).reshape(NPAD * 4)
        out_flat = _SC_KERNEL(fmap, rp)
        nhwc = out_flat[:N_HALF * BIN_ELEMS].reshape(N_HALF, PH * PW, C)
        halves.append(_TC_TRANSPOSE(nhwc))
    out = jnp.concatenate(halves, axis=0)
    return out.reshape(N_ROIS, C, PH, PW)
